# Initial kernel scaffold; baseline (speedup 1.0000x reference)
import jax
import jax.numpy as jnp
from jax.experimental import pallas as pl


def kernel(obj_cls_logits, obj_box_coord, image_sizes):
    raise NotImplementedError("write your pallas kernel here")


# speedup vs baseline: 16.1973x; 16.1973x over previous
import jax
import jax.numpy as jnp
from jax.experimental import pallas as pl
from jax.experimental.pallas import tpu as pltpu

Q = 5000
BK = 128
NB = 40
QP = NB * BK
CR = 151
CP = 152
RCH = 8
NCH = NB // RCH
IOU_T = 0.5


def _prep_body(logits_ref, boxes_ref, sizes_ref, score_ref, vis_ref, planes_ref):
    b = pl.program_id(0)
    t = pl.program_id(1)
    l = logits_ref[0]
    m = jnp.max(l, axis=0, keepdims=True)
    e = jnp.exp(l - m)
    z = jnp.sum(e, axis=0, keepdims=True)
    sm = e / z
    sm0 = sm[0:1, :]
    mx1 = jnp.max(sm[1:, :], axis=0, keepdims=True)
    vis = (mx1 > sm0).astype(jnp.float32)

    lane = jax.lax.broadcasted_iota(jnp.int32, (1, BK), 1)
    real = (t * BK + lane) < Q
    score = jnp.where(real, mx1, -1.0)
    vis = jnp.where(real, vis, 0.0)

    bx = boxes_ref[0]
    cx, cy, w, h = bx[0:1], bx[1:2], bx[2:3], bx[3:4]
    hv = sizes_ref[pl.ds(b, 1), 0:1]
    wv = sizes_ref[pl.ds(b, 1), 1:2]
    x1 = (cx - 0.5 * w) * wv
    y1 = (cy - 0.5 * h) * hv
    x2 = (cx + 0.5 * w) * wv
    y2 = (cy + 0.5 * h) * hv

    score_ref[0, 0, 0, :] = score[0]
    vis_ref[0, 0, 0, :] = vis[0]
    planes_ref[0, :, 0, 0, :] = jnp.concatenate([x1, y1, x2, y2], axis=0)


def _eye():
    li = jax.lax.broadcasted_iota(jnp.int32, (BK, BK), 1)
    si = jax.lax.broadcasted_iota(jnp.int32, (BK, BK), 0)
    return li == si


def _row2col(row, eye):
    return jnp.sum(jnp.where(eye, row, jnp.zeros_like(row)), axis=1, keepdims=True)


def _col2row(col, eye):
    return jnp.sum(jnp.where(eye, col, jnp.zeros_like(col)), axis=0, keepdims=True)


def _main_body(scoresA_ref, visA_ref, planesA_ref, out_ref,
               rank_s, sx_s, area_s, supt_s, keep_s, supp_s):
    eye = _eye()
    lane1 = jax.lax.broadcasted_iota(jnp.int32, (1, BK), 1)

    def rank_blk(t, _):
        srow = scoresA_ref[0, t, :].reshape(1, BK)
        scol = _row2col(srow, eye).reshape(BK, 1, 1)
        icol = (t * BK + jax.lax.broadcasted_iota(
            jnp.int32, (BK, 1, 1), 0))

        def chunk(a, acc):
            sj = scoresA_ref[0, pl.ds(a * RCH, RCH), :].reshape(1, RCH, BK)
            jsub = jax.lax.broadcasted_iota(jnp.int32, (1, RCH, BK), 1)
            jlan = jax.lax.broadcasted_iota(jnp.int32, (1, RCH, BK), 2)
            jidx = (a * RCH + jsub) * BK + jlan
            before = (sj > scol) | ((sj == scol) & (jidx < icol))
            return acc + jnp.sum(before.astype(jnp.int32), axis=(1, 2),
                                 keepdims=True)[:, :, 0]

        rcol = jax.lax.fori_loop(0, NCH, chunk,
                                 jnp.zeros((BK, 1), jnp.int32))
        rank_s[pl.ds(t, 1), :] = _col2row(rcol, eye)
        return 0

    jax.lax.fori_loop(0, NB, rank_blk, 0)

    def perm_blk(t, _):
        pcol = (t * BK + jax.lax.broadcasted_iota(
            jnp.int32, (BK, 1, 1), 0))

        def chunk(a, accs):
            rk = rank_s[pl.ds(a * RCH, RCH), :].reshape(1, RCH, BK)
            onehot = rk == pcol
            outs = []
            for c in range(4):
                v = planesA_ref[0, c, pl.ds(a * RCH, RCH), :].reshape(1, RCH, BK)
                sel = jnp.where(onehot, v, jnp.zeros_like(v))
                outs.append(accs[c] + jnp.sum(sel, axis=(1, 2),
                                              keepdims=True)[:, :, 0])
            return tuple(outs)

        cols = jax.lax.fori_loop(
            0, NCH, chunk,
            tuple(jnp.zeros((BK, 1), jnp.float32) for _ in range(4)))
        for c in range(4):
            sx_s[c, pl.ds(t, 1), :] = _col2row(cols[c], eye)
        ar = (jnp.clip(cols[2] - cols[0], 0.0, None) *
              jnp.clip(cols[3] - cols[1], 0.0, None))
        area_s[pl.ds(t, 1), :] = _col2row(ar, eye)
        return 0

    jax.lax.fori_loop(0, NB, perm_blk, 0)

    supp_s[...] = jnp.zeros((NB, BK), jnp.float32)

    def nms_blk(t, _):
        bx1 = _row2col(sx_s[0, t, :].reshape(1, BK), eye).reshape(BK, 1, 1)
        by1 = _row2col(sx_s[1, t, :].reshape(1, BK), eye).reshape(BK, 1, 1)
        bx2 = _row2col(sx_s[2, t, :].reshape(1, BK), eye).reshape(BK, 1, 1)
        by2 = _row2col(sx_s[3, t, :].reshape(1, BK), eye).reshape(BK, 1, 1)
        bar = _row2col(area_s[t, :].reshape(1, BK), eye).reshape(BK, 1, 1)

        def iou_chunk(a, _c):
            cx1 = sx_s[0, pl.ds(a * RCH, RCH), :].reshape(1, RCH, BK)
            cy1 = sx_s[1, pl.ds(a * RCH, RCH), :].reshape(1, RCH, BK)
            cx2 = sx_s[2, pl.ds(a * RCH, RCH), :].reshape(1, RCH, BK)
            cy2 = sx_s[3, pl.ds(a * RCH, RCH), :].reshape(1, RCH, BK)
            car = area_s[pl.ds(a * RCH, RCH), :].reshape(1, RCH, BK)
            xx1 = jnp.maximum(bx1, cx1)
            yy1 = jnp.maximum(by1, cy1)
            xx2 = jnp.minimum(bx2, cx2)
            yy2 = jnp.minimum(by2, cy2)
            inter = (jnp.clip(xx2 - xx1, 0.0, None) *
                     jnp.clip(yy2 - yy1, 0.0, None))
            union = (bar + car) - inter
            iou = inter / jnp.maximum(union, 1e-9)
            supt_s[:, pl.ds(a * RCH, RCH), :] = (iou > IOU_T).astype(jnp.float32)
            return 0

        jax.lax.fori_loop(0, NCH, iou_chunk, 0)

        keep0 = 1.0 - supp_s[pl.ds(t, 1), :]

        def sweep(i, keep):
            ki = jnp.sum(jnp.where(lane1 == i, keep, jnp.zeros_like(keep)))
            row = supt_s[i, t, :].reshape(1, BK)
            later = (lane1 > i).astype(jnp.float32)
            return keep * (1.0 - ki * row * later)

        keep = jax.lax.fori_loop(0, BK, sweep, keep0)
        keep_s[pl.ds(t, 1), :] = keep
        kcol = _row2col(keep, eye).reshape(BK, 1, 1)

        def cross_chunk(a, _c):
            sup = supt_s[:, pl.ds(a * RCH, RCH), :]
            contrib = jnp.max(sup * kcol, axis=0)
            supp_s[pl.ds(a * RCH, RCH), :] = jnp.maximum(
                supp_s[pl.ds(a * RCH, RCH), :], contrib)
            return 0

        jax.lax.fori_loop(0, NCH, cross_chunk, 0)
        return 0

    jax.lax.fori_loop(0, NB, nms_blk, 0)

    def out_blk(t, _):
        rcol = _row2col(rank_s[t, :].reshape(1, BK).astype(jnp.float32),
                        eye).astype(jnp.int32).reshape(BK, 1, 1)

        def chunk(a, acc):
            ks = keep_s[pl.ds(a * RCH, RCH), :].reshape(1, RCH, BK)
            jsub = jax.lax.broadcasted_iota(jnp.int32, (1, RCH, BK), 1)
            jlan = jax.lax.broadcasted_iota(jnp.int32, (1, RCH, BK), 2)
            pidx = (a * RCH + jsub) * BK + jlan
            sel = jnp.where(pidx == rcol, ks, jnp.zeros_like(ks))
            return acc + jnp.sum(sel, axis=(1, 2), keepdims=True)[:, :, 0]

        kcol = jax.lax.fori_loop(0, NCH, chunk,
                                 jnp.zeros((BK, 1), jnp.float32))
        keep = _col2row(kcol, eye) * visA_ref[0, t, :].reshape(1, BK)
        for c in range(4):
            out_ref[0, c, t, :] = (planesA_ref[0, c, t, :].reshape(1, BK)
                                   * keep)[0]
        out_ref[0, 4, t, :] = (scoresA_ref[0, t, :].reshape(1, BK) * keep)[0]
        return 0

    jax.lax.fori_loop(0, NB, out_blk, 0)


def kernel(obj_cls_logits, obj_box_coord, image_sizes):
    B = obj_cls_logits.shape[0]
    logitsT = jnp.transpose(obj_cls_logits, (0, 2, 1))
    logitsT = jnp.pad(logitsT, ((0, 0), (0, CP - CR), (0, QP - Q)),
                      constant_values=-1e30)
    boxesT = jnp.pad(jnp.transpose(obj_box_coord, (0, 2, 1)),
                     ((0, 0), (0, 0), (0, QP - Q)))
    sizes_f = image_sizes.astype(jnp.float32)

    scores, vis, planes = pl.pallas_call(
        _prep_body,
        grid=(B, NB),
        in_specs=[
            pl.BlockSpec((1, CP, BK), lambda b, t: (b, 0, t)),
            pl.BlockSpec((1, 4, BK), lambda b, t: (b, 0, t)),
            pl.BlockSpec((2, 2), lambda b, t: (0, 0)),
        ],
        out_specs=[
            pl.BlockSpec((1, 1, 1, BK), lambda b, t: (b, t, 0, 0)),
            pl.BlockSpec((1, 1, 1, BK), lambda b, t: (b, t, 0, 0)),
            pl.BlockSpec((1, 4, 1, 1, BK), lambda b, t: (b, 0, t, 0, 0)),
        ],
        out_shape=[
            jax.ShapeDtypeStruct((B, NB, 1, BK), jnp.float32),
            jax.ShapeDtypeStruct((B, NB, 1, BK), jnp.float32),
            jax.ShapeDtypeStruct((B, 4, NB, 1, BK), jnp.float32),
        ],
    )(logitsT, boxesT, sizes_f)
    scores = scores.reshape(B, NB, BK)
    vis = vis.reshape(B, NB, BK)
    planes = planes.reshape(B, 4, NB, BK)

    outA = pl.pallas_call(
        _main_body,
        grid=(B,),
        in_specs=[
            pl.BlockSpec((1, NB, BK), lambda b: (b, 0, 0)),
            pl.BlockSpec((1, NB, BK), lambda b: (b, 0, 0)),
            pl.BlockSpec((1, 4, NB, BK), lambda b: (b, 0, 0, 0)),
        ],
        out_specs=pl.BlockSpec((1, 5, NB, BK), lambda b: (b, 0, 0, 0)),
        out_shape=jax.ShapeDtypeStruct((B, 5, NB, BK), jnp.float32),
        scratch_shapes=[
            pltpu.VMEM((NB, BK), jnp.int32),
            pltpu.VMEM((4, NB, BK), jnp.float32),
            pltpu.VMEM((NB, BK), jnp.float32),
            pltpu.VMEM((BK, NB, BK), jnp.float32),
            pltpu.VMEM((NB, BK), jnp.float32),
            pltpu.VMEM((NB, BK), jnp.float32),
        ],
    )(scores, vis, planes)

    out = jnp.transpose(outA, (0, 2, 3, 1)).reshape(B, QP, 5)[:, :Q, :]
    return out
